# baseline (speedup 1.0000x reference)
import functools

import jax
import jax.numpy as jnp
import numpy as np
from jax import lax
from jax.experimental import pallas as pl
from jax.experimental.pallas import tpu as pltpu
from jax.experimental.pallas import tpu_sc as plsc

ROWS = 128
COLS = 32768
KTOP = 64
NC = 2
NS = 16
L = 16
NW = NC * NS
RPW = ROWS // NW
NVEC = COLS // L
NB = 256
SSTRIDE = 16
SMIN = 16

_MESH = plsc.VectorSubcoreMesh(
    core_axis_name="c", subcore_axis_name="s", num_cores=NC, num_subcores=NS
)

_MASK31 = np.int32(0x7FFFFFFF)


def _keyize(u):
    return u ^ (lax.shift_right_arithmetic(u, 31) & _MASK31)


def _xlane(v, perm):
    return v.at[perm].get(mode="promise_in_bounds")


def _prefix16(m01, iota):
    p = m01
    for k in (1, 2, 4, 8):
        s = _xlane(p, jnp.maximum(iota - k, 0))
        p = p + jnp.where(iota >= k, s, 0)
    return p


def _clean16(v, iota):
    for s in (8, 4, 2, 1):
        p = _xlane(v, iota ^ s)
        take_min = (iota & s) == 0
        v = jnp.where(take_min, jnp.minimum(v, p), jnp.maximum(v, p))
    return v


def _merge16(a, b, iota):
    br = lax.rev(b, (0,))
    lo = jnp.minimum(a, br)
    hi = jnp.maximum(a, br)
    return _clean16(lo, iota), _clean16(hi, iota)


def _sort64(d0, d1, d2, d3, iota):
    s0 = jnp.sort(d0)
    s1 = jnp.sort(d1)
    s2 = jnp.sort(d2)
    s3 = jnp.sort(d3)
    a0, a1 = _merge16(s0, s1, iota)
    b0, b1 = _merge16(s2, s3, iota)
    rb1 = lax.rev(b1, (0,))
    rb0 = lax.rev(b0, (0,))
    l0 = jnp.minimum(a0, rb1)
    l1 = jnp.minimum(a1, rb0)
    h0 = jnp.maximum(a0, rb1)
    h1 = jnp.maximum(a1, rb0)
    e0 = _clean16(jnp.minimum(l0, l1), iota)
    e1 = _clean16(jnp.maximum(l0, l1), iota)
    e2 = _clean16(jnp.minimum(h0, h1), iota)
    e3 = _clean16(jnp.maximum(h0, h1), iota)
    return e0, e1, e2, e3


def _body(x_hbm, out_hbm, xa_v, xb_v, cand_v, hist_v, def_v, out_v, sa, sb):
    wid = lax.axis_index("s") * NC + lax.axis_index("c")
    iota = lax.iota(jnp.int32, L)
    zeros16 = jnp.zeros((L,), jnp.int32)
    ones16 = jnp.ones((L,), jnp.int32)
    last16 = jnp.full((L,), L - 1, jnp.int32)

    def zero_hist():
        for i in range(NB // L):
            hist_v[pl.ds(i * L, L)] = zeros16

    def find_threshold(k):
        def blk(t, carry):
            running, fb, above = carry
            i = NB // L - 1 - t
            bs = jnp.sum(hist_v[pl.ds(i * L, L)])
            hit = jnp.logical_and(fb < 0, running + bs >= k)
            return (
                running + bs,
                jnp.where(hit, i, fb),
                jnp.where(hit, running, above),
            )
        _, fb, above_blk = lax.fori_loop(
            0, NB // L, blk, (jnp.int32(0), jnp.int32(-1), jnp.int32(0))
        )
        h = hist_v[pl.ds(fb * L, L)]
        hr = lax.rev(h, (0,))
        c = plsc.cumsum(hr)
        crossed = (above_blk + c) >= k
        ts = plsc.all_reduce_ffs(crossed)
        bst = fb * L + (L - 1) - ts[0]
        cs = _xlane(c, ts)
        hs = _xlane(hr, ts)
        above = above_blk + cs[0] - hs[0]
        return bst, above

    def refine_level(bucket_fn, count, k, cur_d):
        zero_hist()
        nv = (count + (L - 1)) // L

        def hst(i, c):
            sk = cand_v[pl.ds(i * L, L)]
            valid = (i * L + iota) < count
            plsc.addupdate_scatter(hist_v, [bucket_fn(sk)], ones16, mask=valid)
            return c
        lax.fori_loop(0, nv, hst, 0)

        bst, above = find_threshold(k)

        def flt(i, carry):
            cd, cc = carry
            sk = cand_v[pl.ds(i * L, L)]
            valid = (i * L + iota) < count
            b = bucket_fn(sk)
            gt = jnp.logical_and(valid, b > bst)
            eq = jnp.logical_and(valid, b == bst)
            pg = _prefix16(gt.astype(jnp.int32), iota)
            plsc.store_scatter(def_v, [cd + pg - 1], sk, mask=gt)
            pe = _prefix16(eq.astype(jnp.int32), iota)
            plsc.store_scatter(cand_v, [cc + pe - 1], sk, mask=eq)
            return (cd + _xlane(pg, last16), cc + _xlane(pe, last16))
        cur_d, cur_c = lax.fori_loop(0, nv, flt, (cur_d, zeros16))
        return cur_d, jnp.max(cur_c), bst, above

    def bucket_b1(sk):
        return lax.shift_right_arithmetic(sk, 24) + 128

    def bucket_b2(sk):
        return lax.shift_right_arithmetic(sk, 16) & jnp.int32(0xFF)

    def bucket_b3(sk):
        return lax.shift_right_arithmetic(sk, 8) & jnp.int32(0xFF)

    def bucket_b4(sk):
        return sk & jnp.int32(0xFF)

    def process_row(x_v, r):
        zero_hist()

        def samp(i, c):
            x = x_v[pl.ds(i * (SSTRIDE * L), L)]
            sk = _keyize(lax.bitcast_convert_type(x, jnp.int32))
            plsc.addupdate_scatter(hist_v, [bucket_b1(sk)], ones16)
            return c
        lax.fori_loop(0, NVEC // SSTRIDE, samp, 0)
        bst_s, _ = find_threshold(jnp.int32(SMIN))
        tk = lax.shift_left(bst_s - 128, 24)
        t_low = lax.bitcast_convert_type(
            tk ^ (lax.shift_right_arithmetic(tk, 31) & _MASK31), jnp.float32)

        def compact_pass(t):
            def pb(i, cc):
                for u in range(8):
                    x = x_v[pl.ds((8 * i + u) * L, L)]
                    hot = x >= t
                    p = _prefix16(hot.astype(jnp.int32), iota)
                    plsc.store_scatter(
                        cand_v, [cc + p - 1],
                        lax.bitcast_convert_type(x, jnp.int32), mask=hot)
                    cc = cc + _xlane(p, last16)
                return cc
            return jnp.max(lax.fori_loop(0, NVEC // 8, pb, zeros16))

        count = compact_pass(t_low)
        count = lax.cond(
            count < KTOP,
            lambda: compact_pass(jnp.float32(-jnp.inf)),
            lambda: count,
        )

        def kz(i, c):
            u = cand_v[pl.ds(i * L, L)]
            cand_v[pl.ds(i * L, L)] = _keyize(u)
            return c
        lax.fori_loop(0, (count + (L - 1)) // L, kz, 0)

        cur_d, c1, bs1, above1 = refine_level(
            bucket_b1, count, jnp.int32(KTOP), zeros16)
        k1 = jnp.int32(KTOP) - above1
        cur_d, c2, bs2, above2 = refine_level(bucket_b2, c1, k1, cur_d)
        k2 = k1 - above2
        cur_d, c3, bs3, above3 = refine_level(bucket_b3, c2, k2, cur_d)
        k3 = k2 - above3
        cur_d, _c4, bs4, above4 = refine_level(bucket_b4, c3, k3, cur_d)
        k4 = k3 - above4

        t_key = (
            lax.shift_left(bs1 - 128, 24)
            | lax.shift_left(bs2, 16)
            | lax.shift_left(bs3, 8)
            | bs4
        )
        for t in range(4):
            m = (t * L + iota) < k4
            idx = cur_d + t * L + iota
            plsc.store_scatter(def_v, [idx], jnp.full((L,), t_key), mask=m)

        d0 = def_v[pl.ds(0, L)]
        d1 = def_v[pl.ds(L, L)]
        d2 = def_v[pl.ds(2 * L, L)]
        d3 = def_v[pl.ds(3 * L, L)]
        e0, e1, e2, e3 = _sort64(d0, d1, d2, d3, iota)
        for t, e in enumerate((e3, e2, e1, e0)):
            w = lax.rev(e, (0,))
            u = w ^ (lax.shift_right_arithmetic(w, 31) & _MASK31)
            out_v[pl.ds(t * L, L)] = lax.bitcast_convert_type(u, jnp.float32)
        pltpu.sync_copy(out_v, out_hbm.at[r])

    bufs = (xa_v, xb_v)
    sems = (sa, sb)
    r0 = wid * RPW
    cp = pltpu.async_copy(x_hbm.at[r0], bufs[0], sems[0])
    for j in range(RPW):
        cp.wait()
        if j + 1 < RPW:
            cp = pltpu.async_copy(
                x_hbm.at[r0 + j + 1], bufs[(j + 1) % 2], sems[(j + 1) % 2])
        process_row(bufs[j % 2], r0 + j)


_topk_sc = functools.partial(
    pl.kernel,
    out_type=jax.ShapeDtypeStruct((ROWS, KTOP), jnp.float32),
    mesh=_MESH,
    compiler_params=pltpu.CompilerParams(needs_layout_passes=False),
    scratch_types=[
        pltpu.VMEM((COLS,), jnp.float32),
        pltpu.VMEM((COLS,), jnp.float32),
        pltpu.VMEM((COLS,), jnp.int32),
        pltpu.VMEM((NB,), jnp.int32),
        pltpu.VMEM((2 * KTOP,), jnp.int32),
        pltpu.VMEM((KTOP,), jnp.float32),
        pltpu.SemaphoreType.DMA,
        pltpu.SemaphoreType.DMA,
    ],
)(_body)


def kernel(input):
    return _topk_sc(input)


# speedup vs baseline: 3.0715x; 3.0715x over previous
import functools

import jax
import jax.numpy as jnp
import numpy as np
from jax import lax
from jax.experimental import pallas as pl
from jax.experimental.pallas import tpu as pltpu
from jax.experimental.pallas import tpu_sc as plsc

ROWS = 128
COLS = 32768
KTOP = 64
NC = 2
NS = 16
L = 16
NW = NC * NS
RPW = ROWS // NW
NVEC = COLS // L
NB = 256
SSTRIDE = 16
SMIN = 16

_MESH = plsc.VectorSubcoreMesh(
    core_axis_name="c", subcore_axis_name="s", num_cores=NC, num_subcores=NS
)

_MASK31 = np.int32(0x7FFFFFFF)


def _keyize(u):
    return u ^ (lax.shift_right_arithmetic(u, 31) & _MASK31)


def _xlane(v, perm):
    return v.at[perm].get(mode="promise_in_bounds")


def _prefix16(m01, iota):
    p = m01
    for k in (1, 2, 4, 8):
        s = _xlane(p, jnp.maximum(iota - k, 0))
        p = p + jnp.where(iota >= k, s, 0)
    return p


def _clean16(v, iota):
    for s in (8, 4, 2, 1):
        p = _xlane(v, iota ^ s)
        take_min = (iota & s) == 0
        v = jnp.where(take_min, jnp.minimum(v, p), jnp.maximum(v, p))
    return v


def _merge16(a, b, iota):
    br = lax.rev(b, (0,))
    lo = jnp.minimum(a, br)
    hi = jnp.maximum(a, br)
    return _clean16(lo, iota), _clean16(hi, iota)


def _sort64(d0, d1, d2, d3, iota):
    s0 = jnp.sort(d0)
    s1 = jnp.sort(d1)
    s2 = jnp.sort(d2)
    s3 = jnp.sort(d3)
    a0, a1 = _merge16(s0, s1, iota)
    b0, b1 = _merge16(s2, s3, iota)
    rb1 = lax.rev(b1, (0,))
    rb0 = lax.rev(b0, (0,))
    l0 = jnp.minimum(a0, rb1)
    l1 = jnp.minimum(a1, rb0)
    h0 = jnp.maximum(a0, rb1)
    h1 = jnp.maximum(a1, rb0)
    e0 = _clean16(jnp.minimum(l0, l1), iota)
    e1 = _clean16(jnp.maximum(l0, l1), iota)
    e2 = _clean16(jnp.minimum(h0, h1), iota)
    e3 = _clean16(jnp.maximum(h0, h1), iota)
    return e0, e1, e2, e3


def _body(x_hbm, out_hbm, xa_v, xb_v, cand_v, hist_v, def_v, out_v, sa, sb):
    wid = lax.axis_index("s") * NC + lax.axis_index("c")
    iota = lax.iota(jnp.int32, L)
    zeros16 = jnp.zeros((L,), jnp.int32)
    ones16 = jnp.ones((L,), jnp.int32)
    last16 = jnp.full((L,), L - 1, jnp.int32)

    def zero_hist():
        for i in range(NB // L):
            hist_v[pl.ds(i * L, L)] = zeros16

    def find_threshold(k):
        def blk(t, carry):
            running, fb, above = carry
            i = NB // L - 1 - t
            bs = jnp.sum(hist_v[pl.ds(i * L, L)])
            hit = jnp.logical_and(fb < 0, running + bs >= k)
            return (
                running + bs,
                jnp.where(hit, i, fb),
                jnp.where(hit, running, above),
            )
        _, fb, above_blk = lax.fori_loop(
            0, NB // L, blk, (jnp.int32(0), jnp.int32(-1), jnp.int32(0))
        )
        h = hist_v[pl.ds(fb * L, L)]
        hr = lax.rev(h, (0,))
        c = plsc.cumsum(hr)
        crossed = (above_blk + c) >= k
        ts = plsc.all_reduce_ffs(crossed)
        bst = fb * L + (L - 1) - ts[0]
        cs = _xlane(c, ts)
        hs = _xlane(hr, ts)
        above = above_blk + cs[0] - hs[0]
        return bst, above

    def refine_level(bucket_fn, count, k, cur_d):
        zero_hist()
        nv = (count + (L - 1)) // L

        def hst(i, c):
            sk = cand_v[pl.ds(i * L, L)]
            valid = (i * L + iota) < count
            plsc.addupdate_scatter(hist_v, [bucket_fn(sk)], ones16, mask=valid)
            return c
        lax.fori_loop(0, nv, hst, 0)

        bst, above = find_threshold(k)

        def flt(i, carry):
            cd, cc = carry
            sk = cand_v[pl.ds(i * L, L)]
            valid = (i * L + iota) < count
            b = bucket_fn(sk)
            gt = jnp.logical_and(valid, b > bst)
            eq = jnp.logical_and(valid, b == bst)
            pg = _prefix16(gt.astype(jnp.int32), iota)
            plsc.store_scatter(def_v, [cd + pg - 1], sk, mask=gt)
            pe = _prefix16(eq.astype(jnp.int32), iota)
            plsc.store_scatter(cand_v, [cc + pe - 1], sk, mask=eq)
            return (cd + _xlane(pg, last16), cc + _xlane(pe, last16))
        cur_d, cur_c = lax.fori_loop(0, nv, flt, (cur_d, zeros16))
        return cur_d, jnp.max(cur_c), bst, above

    def bucket_b1(sk):
        return lax.shift_right_arithmetic(sk, 24) + 128

    def bucket_b2(sk):
        return lax.shift_right_arithmetic(sk, 16) & jnp.int32(0xFF)

    def bucket_b3(sk):
        return lax.shift_right_arithmetic(sk, 8) & jnp.int32(0xFF)

    def bucket_b4(sk):
        return sk & jnp.int32(0xFF)

    def process_row(x_v, r):
        zero_hist()

        @plsc.parallel_loop(0, NVEC // SSTRIDE, step=1, unroll=4)
        def samp(i):
            x = x_v[pl.ds(i * (SSTRIDE * L), L)]
            sk = _keyize(lax.bitcast_convert_type(x, jnp.int32))
            plsc.addupdate_scatter(hist_v, [bucket_b1(sk)], ones16)
        bst_s, _ = find_threshold(jnp.int32(SMIN))
        tk = lax.shift_left(bst_s - 128, 24)
        t_low = lax.bitcast_convert_type(
            tk ^ (lax.shift_right_arithmetic(tk, 31) & _MASK31), jnp.float32)

        def compact_pass(t):
            @plsc.parallel_loop(0, NVEC, step=1, unroll=8, carry=zeros16)
            def pb(i, cc):
                x = x_v[pl.ds(i * L, L)]
                hot = x >= t
                p = plsc.cumsum(hot.astype(jnp.int32))
                plsc.store_scatter(
                    cand_v, [cc + p - 1],
                    lax.bitcast_convert_type(x, jnp.int32), mask=hot)
                return cc + plsc.all_reduce_population_count(hot)
            return jnp.max(pb)

        count = compact_pass(t_low)
        count = lax.cond(
            count < KTOP,
            lambda: compact_pass(jnp.float32(-jnp.inf)),
            lambda: count,
        )

        def kz(i, c):
            u = cand_v[pl.ds(i * L, L)]
            cand_v[pl.ds(i * L, L)] = _keyize(u)
            return c
        lax.fori_loop(0, (count + (L - 1)) // L, kz, 0)

        cur_d, c1, bs1, above1 = refine_level(
            bucket_b1, count, jnp.int32(KTOP), zeros16)
        k1 = jnp.int32(KTOP) - above1
        cur_d, c2, bs2, above2 = refine_level(bucket_b2, c1, k1, cur_d)
        k2 = k1 - above2
        cur_d, c3, bs3, above3 = refine_level(bucket_b3, c2, k2, cur_d)
        k3 = k2 - above3
        cur_d, _c4, bs4, above4 = refine_level(bucket_b4, c3, k3, cur_d)
        k4 = k3 - above4

        t_key = (
            lax.shift_left(bs1 - 128, 24)
            | lax.shift_left(bs2, 16)
            | lax.shift_left(bs3, 8)
            | bs4
        )
        for t in range(4):
            m = (t * L + iota) < k4
            idx = cur_d + t * L + iota
            plsc.store_scatter(def_v, [idx], jnp.full((L,), t_key), mask=m)

        d0 = def_v[pl.ds(0, L)]
        d1 = def_v[pl.ds(L, L)]
        d2 = def_v[pl.ds(2 * L, L)]
        d3 = def_v[pl.ds(3 * L, L)]
        e0, e1, e2, e3 = _sort64(d0, d1, d2, d3, iota)
        for t, e in enumerate((e3, e2, e1, e0)):
            w = lax.rev(e, (0,))
            u = w ^ (lax.shift_right_arithmetic(w, 31) & _MASK31)
            out_v[pl.ds(t * L, L)] = lax.bitcast_convert_type(u, jnp.float32)
        pltpu.sync_copy(out_v, out_hbm.at[r])

    bufs = (xa_v, xb_v)
    sems = (sa, sb)
    r0 = wid * RPW
    cp = pltpu.async_copy(x_hbm.at[r0], bufs[0], sems[0])
    for j in range(RPW):
        cp.wait()
        if j + 1 < RPW:
            cp = pltpu.async_copy(
                x_hbm.at[r0 + j + 1], bufs[(j + 1) % 2], sems[(j + 1) % 2])
        process_row(bufs[j % 2], r0 + j)


_topk_sc = functools.partial(
    pl.kernel,
    out_type=jax.ShapeDtypeStruct((ROWS, KTOP), jnp.float32),
    mesh=_MESH,
    compiler_params=pltpu.CompilerParams(needs_layout_passes=False),
    scratch_types=[
        pltpu.VMEM((COLS,), jnp.float32),
        pltpu.VMEM((COLS,), jnp.float32),
        pltpu.VMEM((COLS,), jnp.int32),
        pltpu.VMEM((NB,), jnp.int32),
        pltpu.VMEM((2 * KTOP,), jnp.int32),
        pltpu.VMEM((KTOP,), jnp.float32),
        pltpu.SemaphoreType.DMA,
        pltpu.SemaphoreType.DMA,
    ],
)(_body)


def kernel(input):
    return _topk_sc(input)


# speedup vs baseline: 3.6018x; 1.1727x over previous
import functools

import jax
import jax.numpy as jnp
import numpy as np
from jax import lax
from jax.experimental import pallas as pl
from jax.experimental.pallas import tpu as pltpu
from jax.experimental.pallas import tpu_sc as plsc

ROWS = 128
COLS = 32768
KTOP = 64
NC = 2
NS = 16
L = 16
NW = NC * NS
RPW = ROWS // NW
NVEC = COLS // L
NB = 256
SSTRIDE = 16
SMIN = 16

_MESH = plsc.VectorSubcoreMesh(
    core_axis_name="c", subcore_axis_name="s", num_cores=NC, num_subcores=NS
)

_MASK31 = np.int32(0x7FFFFFFF)


def _keyize(u):
    return u ^ (lax.shift_right_arithmetic(u, 31) & _MASK31)


def _xlane(v, perm):
    return v.at[perm].get(mode="promise_in_bounds")


def _prefix16(m01, iota):
    p = m01
    for k in (1, 2, 4, 8):
        s = _xlane(p, jnp.maximum(iota - k, 0))
        p = p + jnp.where(iota >= k, s, 0)
    return p


def _clean16(v, iota):
    for s in (8, 4, 2, 1):
        p = _xlane(v, iota ^ s)
        take_min = (iota & s) == 0
        v = jnp.where(take_min, jnp.minimum(v, p), jnp.maximum(v, p))
    return v


def _merge16(a, b, iota):
    br = lax.rev(b, (0,))
    lo = jnp.minimum(a, br)
    hi = jnp.maximum(a, br)
    return _clean16(lo, iota), _clean16(hi, iota)


def _sort64(d0, d1, d2, d3, iota):
    s0 = jnp.sort(d0)
    s1 = jnp.sort(d1)
    s2 = jnp.sort(d2)
    s3 = jnp.sort(d3)
    a0, a1 = _merge16(s0, s1, iota)
    b0, b1 = _merge16(s2, s3, iota)
    rb1 = lax.rev(b1, (0,))
    rb0 = lax.rev(b0, (0,))
    l0 = jnp.minimum(a0, rb1)
    l1 = jnp.minimum(a1, rb0)
    h0 = jnp.maximum(a0, rb1)
    h1 = jnp.maximum(a1, rb0)
    e0 = _clean16(jnp.minimum(l0, l1), iota)
    e1 = _clean16(jnp.maximum(l0, l1), iota)
    e2 = _clean16(jnp.minimum(h0, h1), iota)
    e3 = _clean16(jnp.maximum(h0, h1), iota)
    return e0, e1, e2, e3


def _body(x_hbm, out_hbm, xa_v, xb_v, cand_v, hist_v, def_v, out_v, sa, sb):
    wid = lax.axis_index("s") * NC + lax.axis_index("c")
    iota = lax.iota(jnp.int32, L)
    zeros16 = jnp.zeros((L,), jnp.int32)
    ones16 = jnp.ones((L,), jnp.int32)
    last16 = jnp.full((L,), L - 1, jnp.int32)

    def zero_hist():
        for i in range(NB // L):
            hist_v[pl.ds(i * L, L)] = zeros16

    def find_threshold(k):
        @plsc.parallel_loop(0, NB // L, step=1, unroll=4, carry=zeros16)
        def bsums(i, acc):
            c = plsc.cumsum(hist_v[pl.ds(i * L, L)])
            return acc + jnp.where(iota == i, _xlane(c, last16), 0)
        br = lax.rev(bsums, (0,))
        cb = plsc.cumsum(br)
        tb = plsc.all_reduce_ffs(cb >= k)
        fb = (NB // L - 1) - tb[0]
        above_blk = _xlane(cb, tb)[0] - _xlane(br, tb)[0]
        h = hist_v[pl.ds(fb * L, L)]
        hr = lax.rev(h, (0,))
        c = plsc.cumsum(hr)
        crossed = (above_blk + c) >= k
        ts = plsc.all_reduce_ffs(crossed)
        bst = fb * L + (L - 1) - ts[0]
        above = above_blk + _xlane(c, ts)[0] - _xlane(hr, ts)[0]
        return bst, above

    def refine_level(bucket_fn, count, k, cur_d):
        zero_hist()
        nv = (count + (L - 1)) // L

        @plsc.parallel_loop(0, nv, step=1, unroll=2)
        def hst(i):
            sk = cand_v[pl.ds(i * L, L)]
            valid = (i * L + iota) < count
            plsc.addupdate_scatter(hist_v, [bucket_fn(sk)], ones16, mask=valid)

        bst, above = find_threshold(k)

        @plsc.parallel_loop(0, nv, step=1, unroll=2, carry=(cur_d, zeros16))
        def flt(i, carry):
            cd, cc = carry
            sk = cand_v[pl.ds(i * L, L)]
            valid = (i * L + iota) < count
            b = bucket_fn(sk)
            gt = jnp.logical_and(valid, b > bst)
            eq = jnp.logical_and(valid, b == bst)
            pg = plsc.cumsum(gt.astype(jnp.int32))
            plsc.store_scatter(def_v, [cd + pg - 1], sk, mask=gt)
            pe = plsc.cumsum(eq.astype(jnp.int32))
            plsc.store_scatter(cand_v, [cc + pe - 1], sk, mask=eq)
            return (cd + _xlane(pg, last16), cc + _xlane(pe, last16))
        cur_d, cur_c = flt
        return cur_d, jnp.max(cur_c), bst, above

    def bucket_b1(sk):
        return lax.shift_right_arithmetic(sk, 24) + 128

    def bucket_b2(sk):
        return lax.shift_right_arithmetic(sk, 16) & jnp.int32(0xFF)

    def bucket_b3(sk):
        return lax.shift_right_arithmetic(sk, 8) & jnp.int32(0xFF)

    def bucket_b4(sk):
        return sk & jnp.int32(0xFF)

    def process_row(x_v, r):
        zero_hist()

        @plsc.parallel_loop(0, NVEC // SSTRIDE, step=1, unroll=4)
        def samp(i):
            x = x_v[pl.ds(i * (SSTRIDE * L), L)]
            sk = _keyize(lax.bitcast_convert_type(x, jnp.int32))
            plsc.addupdate_scatter(hist_v, [bucket_b1(sk)], ones16)
        bst_s, _ = find_threshold(jnp.int32(SMIN))
        tk = lax.shift_left(bst_s - 128, 24)
        t_low = lax.bitcast_convert_type(
            tk ^ (lax.shift_right_arithmetic(tk, 31) & _MASK31), jnp.float32)

        def compact_pass(t):
            @plsc.parallel_loop(0, NVEC, step=1, unroll=8, carry=zeros16)
            def pb(i, cc):
                x = x_v[pl.ds(i * L, L)]
                hot = x >= t
                p = plsc.cumsum(hot.astype(jnp.int32))
                plsc.store_scatter(
                    cand_v, [cc + p - 1],
                    lax.bitcast_convert_type(x, jnp.int32), mask=hot)
                return cc + plsc.all_reduce_population_count(hot)
            return jnp.max(pb)

        count = compact_pass(t_low)
        count = lax.cond(
            count < KTOP,
            lambda: compact_pass(jnp.float32(-jnp.inf)),
            lambda: count,
        )

        @plsc.parallel_loop(0, (count + (L - 1)) // L, step=1, unroll=4)
        def kz(i):
            u = cand_v[pl.ds(i * L, L)]
            cand_v[pl.ds(i * L, L)] = _keyize(u)

        cur_d, c1, bs1, above1 = refine_level(
            bucket_b1, count, jnp.int32(KTOP), zeros16)
        k1 = jnp.int32(KTOP) - above1
        cur_d, c2, bs2, above2 = refine_level(bucket_b2, c1, k1, cur_d)
        k2 = k1 - above2
        cur_d, c3, bs3, above3 = refine_level(bucket_b3, c2, k2, cur_d)
        k3 = k2 - above3
        cur_d, _c4, bs4, above4 = refine_level(bucket_b4, c3, k3, cur_d)
        k4 = k3 - above4

        t_key = (
            lax.shift_left(bs1 - 128, 24)
            | lax.shift_left(bs2, 16)
            | lax.shift_left(bs3, 8)
            | bs4
        )
        for t in range(4):
            m = (t * L + iota) < k4
            idx = cur_d + t * L + iota
            plsc.store_scatter(def_v, [idx], jnp.full((L,), t_key), mask=m)

        d0 = def_v[pl.ds(0, L)]
        d1 = def_v[pl.ds(L, L)]
        d2 = def_v[pl.ds(2 * L, L)]
        d3 = def_v[pl.ds(3 * L, L)]
        e0, e1, e2, e3 = _sort64(d0, d1, d2, d3, iota)
        for t, e in enumerate((e3, e2, e1, e0)):
            w = lax.rev(e, (0,))
            u = w ^ (lax.shift_right_arithmetic(w, 31) & _MASK31)
            out_v[pl.ds(t * L, L)] = lax.bitcast_convert_type(u, jnp.float32)
        pltpu.sync_copy(out_v, out_hbm.at[r])

    bufs = (xa_v, xb_v)
    sems = (sa, sb)
    r0 = wid * RPW
    cp = pltpu.async_copy(x_hbm.at[r0], bufs[0], sems[0])
    for j in range(RPW):
        cp.wait()
        if j + 1 < RPW:
            cp = pltpu.async_copy(
                x_hbm.at[r0 + j + 1], bufs[(j + 1) % 2], sems[(j + 1) % 2])
        process_row(bufs[j % 2], r0 + j)


_topk_sc = functools.partial(
    pl.kernel,
    out_type=jax.ShapeDtypeStruct((ROWS, KTOP), jnp.float32),
    mesh=_MESH,
    compiler_params=pltpu.CompilerParams(needs_layout_passes=False),
    scratch_types=[
        pltpu.VMEM((COLS,), jnp.float32),
        pltpu.VMEM((COLS,), jnp.float32),
        pltpu.VMEM((COLS,), jnp.int32),
        pltpu.VMEM((NB,), jnp.int32),
        pltpu.VMEM((2 * KTOP,), jnp.int32),
        pltpu.VMEM((KTOP,), jnp.float32),
        pltpu.SemaphoreType.DMA,
        pltpu.SemaphoreType.DMA,
    ],
)(_body)


def kernel(input):
    return _topk_sc(input)


# speedup vs baseline: 4.5271x; 1.2569x over previous
import functools

import jax
import jax.numpy as jnp
import numpy as np
from jax import lax
from jax.experimental import pallas as pl
from jax.experimental.pallas import tpu as pltpu
from jax.experimental.pallas import tpu_sc as plsc

ROWS = 128
COLS = 32768
KTOP = 64
NC = 2
NS = 16
L = 16
NW = NC * NS
RPW = ROWS // NW
NVEC = COLS // L
NB = 256
SSTRIDE = 16
SMIN = 16

_MESH = plsc.VectorSubcoreMesh(
    core_axis_name="c", subcore_axis_name="s", num_cores=NC, num_subcores=NS
)

_MASK31 = np.int32(0x7FFFFFFF)


def _keyize(u):
    return u ^ (lax.shift_right_arithmetic(u, 31) & _MASK31)


def _xlane(v, perm):
    return v.at[perm].get(mode="promise_in_bounds")


def _prefix16(m01, iota):
    p = m01
    for k in (1, 2, 4, 8):
        s = _xlane(p, jnp.maximum(iota - k, 0))
        p = p + jnp.where(iota >= k, s, 0)
    return p


def _clean16(v, iota):
    for s in (8, 4, 2, 1):
        p = _xlane(v, iota ^ s)
        take_min = (iota & s) == 0
        v = jnp.where(take_min, jnp.minimum(v, p), jnp.maximum(v, p))
    return v


def _merge16(a, b, iota):
    br = lax.rev(b, (0,))
    lo = jnp.minimum(a, br)
    hi = jnp.maximum(a, br)
    return _clean16(lo, iota), _clean16(hi, iota)


def _sort64(d0, d1, d2, d3, iota):
    s0 = jnp.sort(d0)
    s1 = jnp.sort(d1)
    s2 = jnp.sort(d2)
    s3 = jnp.sort(d3)
    a0, a1 = _merge16(s0, s1, iota)
    b0, b1 = _merge16(s2, s3, iota)
    rb1 = lax.rev(b1, (0,))
    rb0 = lax.rev(b0, (0,))
    l0 = jnp.minimum(a0, rb1)
    l1 = jnp.minimum(a1, rb0)
    h0 = jnp.maximum(a0, rb1)
    h1 = jnp.maximum(a1, rb0)
    e0 = _clean16(jnp.minimum(l0, l1), iota)
    e1 = _clean16(jnp.maximum(l0, l1), iota)
    e2 = _clean16(jnp.minimum(h0, h1), iota)
    e3 = _clean16(jnp.maximum(h0, h1), iota)
    return e0, e1, e2, e3


def _body(x_hbm, out_hbm, xa_v, xb_v, cand_v, hist_v, def_v, out_v, sa, sb, so):
    wid = lax.axis_index("s") * NC + lax.axis_index("c")
    iota = lax.iota(jnp.int32, L)
    zeros16 = jnp.zeros((L,), jnp.int32)
    ones16 = jnp.ones((L,), jnp.int32)
    last16 = jnp.full((L,), L - 1, jnp.int32)

    def zero_hist():
        for i in range(NB // L):
            hist_v[pl.ds(i * L, L)] = zeros16

    def find_threshold(k):
        @plsc.parallel_loop(0, NB // L, step=1, unroll=4, carry=zeros16)
        def bsums(i, acc):
            c = plsc.cumsum(hist_v[pl.ds(i * L, L)])
            return acc + jnp.where(iota == i, _xlane(c, last16), 0)
        br = lax.rev(bsums, (0,))
        cb = plsc.cumsum(br)
        tb = plsc.all_reduce_ffs(cb >= k)
        fb = (NB // L - 1) - tb[0]
        above_blk = _xlane(cb, tb)[0] - _xlane(br, tb)[0]
        h = hist_v[pl.ds(fb * L, L)]
        hr = lax.rev(h, (0,))
        c = plsc.cumsum(hr)
        crossed = (above_blk + c) >= k
        ts = plsc.all_reduce_ffs(crossed)
        bst = fb * L + (L - 1) - ts[0]
        above = above_blk + _xlane(c, ts)[0] - _xlane(hr, ts)[0]
        return bst, above

    def refine_level(bucket_fn, count, k, cur_d):
        zero_hist()
        nv = (count + (L - 1)) // L

        @plsc.parallel_loop(0, nv, step=1, unroll=2)
        def hst(i):
            sk = cand_v[pl.ds(i * L, L)]
            valid = (i * L + iota) < count
            plsc.addupdate_scatter(hist_v, [bucket_fn(sk)], ones16, mask=valid)

        bst, above = find_threshold(k)

        @plsc.parallel_loop(0, nv, step=1, unroll=2, carry=(cur_d - 1, zeros16 - 1))
        def flt(i, carry):
            cd, cc = carry
            sk = cand_v[pl.ds(i * L, L)]
            valid = (i * L + iota) < count
            b = bucket_fn(sk)
            gt = jnp.logical_and(valid, b > bst)
            eq = jnp.logical_and(valid, b == bst)
            pg = plsc.cumsum(ones16, mask=gt)
            plsc.store_scatter(def_v, [cd + pg], sk, mask=gt)
            pe = plsc.cumsum(ones16, mask=eq)
            plsc.store_scatter(cand_v, [cc + pe], sk, mask=eq)
            return (
                cd + plsc.all_reduce_population_count(gt),
                cc + plsc.all_reduce_population_count(eq),
            )
        cur_d, cur_c = flt
        return cur_d + 1, jnp.max(cur_c) + 1, bst, above

    def bucket_b1(sk):
        return lax.shift_right_arithmetic(sk, 24) + 128

    def bucket_b2(sk):
        return lax.shift_right_arithmetic(sk, 16) & jnp.int32(0xFF)

    def process_row(x_v, r, j):
        zero_hist()

        @plsc.parallel_loop(0, NVEC // SSTRIDE, step=1, unroll=4)
        def samp(i):
            x = x_v[pl.ds(i * (SSTRIDE * L), L)]
            sk = _keyize(lax.bitcast_convert_type(x, jnp.int32))
            plsc.addupdate_scatter(hist_v, [bucket_b1(sk)], ones16)
        bst_s, above_s = find_threshold(jnp.int32(SMIN))
        zero_hist()

        @plsc.parallel_loop(0, NVEC // SSTRIDE, step=1, unroll=4)
        def samp2(i):
            x = x_v[pl.ds(i * (SSTRIDE * L), L)]
            sk = _keyize(lax.bitcast_convert_type(x, jnp.int32))
            m = bucket_b1(sk) == bst_s
            plsc.addupdate_scatter(hist_v, [bucket_b2(sk)], ones16, mask=m)
        bst2_s, _ = find_threshold(jnp.int32(SMIN) - above_s)
        tk = lax.shift_left(bst_s - 128, 24) | lax.shift_left(bst2_s, 16)
        t_low = lax.bitcast_convert_type(
            tk ^ (lax.shift_right_arithmetic(tk, 31) & _MASK31), jnp.float32)

        def compact_pass(t):
            @plsc.parallel_loop(0, NVEC, step=1, unroll=8, carry=zeros16 - 1)
            def pb(i, cc):
                x = x_v[pl.ds(i * L, L)]
                hot = x >= t
                p = plsc.cumsum(ones16, mask=hot)
                plsc.store_scatter(
                    cand_v, [cc + p],
                    lax.bitcast_convert_type(x, jnp.int32), mask=hot)
                return cc + plsc.all_reduce_population_count(hot)
            return jnp.max(pb) + 1

        def trip(s, carry):
            count, t = carry
            need = jnp.logical_or(s == 0, count < KTOP)
            count = lax.cond(need, lambda: compact_pass(t), lambda: count)
            return (count, jnp.float32(-jnp.inf))
        count, _ = lax.fori_loop(0, 2, trip, (jnp.int32(0), t_low))

        @plsc.parallel_loop(0, (count + (L - 1)) // L, step=1, unroll=4)
        def kz(i):
            u = cand_v[pl.ds(i * L, L)]
            cand_v[pl.ds(i * L, L)] = _keyize(u)

        def lvl_body(lvl, carry):
            cur_d, cnt, k, tacc = carry
            shift = 24 - 8 * lvl
            bias = jnp.where(lvl == 0, jnp.int32(0x80), jnp.int32(0))

            def bucket_fn(sk):
                return (
                    lax.shift_right_arithmetic(sk, shift) & jnp.int32(0xFF)
                ) ^ bias
            cur_d, cnt2, bst, above = refine_level(bucket_fn, cnt, k, cur_d)
            tacc = tacc | lax.shift_left(bst ^ bias, shift)
            return (cur_d, cnt2, k - above, tacc)
        cur_d, _cn, k4, t_key = lax.fori_loop(
            0, 4, lvl_body,
            (zeros16, count, jnp.int32(KTOP), jnp.int32(0)))
        for t in range(4):
            m = (t * L + iota) < k4
            idx = cur_d + t * L + iota
            plsc.store_scatter(def_v, [idx], jnp.full((L,), t_key), mask=m)

        d0 = def_v[pl.ds(0, L)]
        d1 = def_v[pl.ds(L, L)]
        d2 = def_v[pl.ds(2 * L, L)]
        d3 = def_v[pl.ds(3 * L, L)]
        e0, e1, e2, e3 = _sort64(d0, d1, d2, d3, iota)
        for t, e in enumerate((e3, e2, e1, e0)):
            w = lax.rev(e, (0,))
            u = w ^ (lax.shift_right_arithmetic(w, 31) & _MASK31)
            out_v[j, pl.ds(t * L, L)] = lax.bitcast_convert_type(
                u, jnp.float32)
        pltpu.async_copy(out_v.at[j], out_hbm.at[r], so)

    r0 = wid * RPW
    nhalf = RPW // 2
    pltpu.async_copy(x_hbm.at[r0], xa_v, sa)
    pltpu.async_copy(x_hbm.at[r0 + 1], xb_v, sb)

    def rows(jo, c):
        r = r0 + 2 * jo
        pltpu.make_async_copy(x_hbm.at[r], xa_v, sa).wait()
        process_row(xa_v, r, 2 * jo)

        @pl.when(jo < nhalf - 1)
        def _():
            pltpu.async_copy(x_hbm.at[r + 2], xa_v, sa)
        pltpu.make_async_copy(x_hbm.at[r + 1], xb_v, sb).wait()
        process_row(xb_v, r + 1, 2 * jo + 1)

        @pl.when(jo < nhalf - 1)
        def _():
            pltpu.async_copy(x_hbm.at[r + 3], xb_v, sb)
        return c
    lax.fori_loop(0, nhalf, rows, 0)
    for _ in range(RPW):
        pltpu.make_async_copy(out_v.at[0], out_hbm.at[r0], so).wait()


_topk_sc = functools.partial(
    pl.kernel,
    out_type=jax.ShapeDtypeStruct((ROWS, KTOP), jnp.float32),
    mesh=_MESH,
    compiler_params=pltpu.CompilerParams(needs_layout_passes=False),
    scratch_types=[
        pltpu.VMEM((COLS,), jnp.float32),
        pltpu.VMEM((COLS,), jnp.float32),
        pltpu.VMEM((COLS,), jnp.int32),
        pltpu.VMEM((NB,), jnp.int32),
        pltpu.VMEM((2 * KTOP,), jnp.int32),
        pltpu.VMEM((RPW, KTOP), jnp.float32),
        pltpu.SemaphoreType.DMA,
        pltpu.SemaphoreType.DMA,
        pltpu.SemaphoreType.DMA,
    ],
)(_body)


def kernel(input):
    return _topk_sc(input)


# speedup vs baseline: 4.5914x; 1.0142x over previous
import functools

import jax
import jax.numpy as jnp
import numpy as np
from jax import lax
from jax.experimental import pallas as pl
from jax.experimental.pallas import tpu as pltpu
from jax.experimental.pallas import tpu_sc as plsc

ROWS = 128
COLS = 32768
KTOP = 64
NC = 2
NS = 16
L = 16
NW = NC * NS
RPW = ROWS // NW
NVEC = COLS // L
NB = 256
SSTRIDE = 16
SMIN = 16

_MESH = plsc.VectorSubcoreMesh(
    core_axis_name="c", subcore_axis_name="s", num_cores=NC, num_subcores=NS
)

_MASK31 = np.int32(0x7FFFFFFF)


def _keyize(u):
    return u ^ (lax.shift_right_arithmetic(u, 31) & _MASK31)


def _xlane(v, perm):
    return v.at[perm].get(mode="promise_in_bounds")


def _prefix16(m01, iota):
    p = m01
    for k in (1, 2, 4, 8):
        s = _xlane(p, jnp.maximum(iota - k, 0))
        p = p + jnp.where(iota >= k, s, 0)
    return p


def _clean16(v, iota):
    for s in (8, 4, 2, 1):
        p = _xlane(v, iota ^ s)
        take_min = (iota & s) == 0
        v = jnp.where(take_min, jnp.minimum(v, p), jnp.maximum(v, p))
    return v


def _merge16(a, b, iota):
    br = lax.rev(b, (0,))
    lo = jnp.minimum(a, br)
    hi = jnp.maximum(a, br)
    return _clean16(lo, iota), _clean16(hi, iota)


def _sort64(d0, d1, d2, d3, iota):
    s0 = jnp.sort(d0)
    s1 = jnp.sort(d1)
    s2 = jnp.sort(d2)
    s3 = jnp.sort(d3)
    a0, a1 = _merge16(s0, s1, iota)
    b0, b1 = _merge16(s2, s3, iota)
    rb1 = lax.rev(b1, (0,))
    rb0 = lax.rev(b0, (0,))
    l0 = jnp.minimum(a0, rb1)
    l1 = jnp.minimum(a1, rb0)
    h0 = jnp.maximum(a0, rb1)
    h1 = jnp.maximum(a1, rb0)
    e0 = _clean16(jnp.minimum(l0, l1), iota)
    e1 = _clean16(jnp.maximum(l0, l1), iota)
    e2 = _clean16(jnp.minimum(h0, h1), iota)
    e3 = _clean16(jnp.maximum(h0, h1), iota)
    return e0, e1, e2, e3


def _body(x_hbm, out_hbm, xa_v, xb_v, cand_v, hist_v, def_v, out_v, sa, sb, so):
    wid = lax.axis_index("s") * NC + lax.axis_index("c")
    iota = lax.iota(jnp.int32, L)
    zeros16 = jnp.zeros((L,), jnp.int32)
    ones16 = jnp.ones((L,), jnp.int32)
    last16 = jnp.full((L,), L - 1, jnp.int32)

    def zero_hist():
        for i in range(NB // L):
            hist_v[pl.ds(i * L, L)] = zeros16

    def find_threshold(k):
        @plsc.parallel_loop(0, NB // L, step=1, unroll=4, carry=zeros16)
        def bsums(i, acc):
            c = plsc.cumsum(hist_v[pl.ds(i * L, L)])
            return acc + jnp.where(iota == i, _xlane(c, last16), 0)
        br = lax.rev(bsums, (0,))
        cb = plsc.cumsum(br)
        tb = plsc.all_reduce_ffs(cb >= k)
        fb = (NB // L - 1) - tb
        above_blk = _xlane(cb, tb) - _xlane(br, tb)
        h = plsc.load_gather(hist_v, [fb * L + iota])
        hr = lax.rev(h, (0,))
        c = plsc.cumsum(hr)
        crossed = (above_blk + c) >= k
        ts = plsc.all_reduce_ffs(crossed)
        bst = fb * L + (L - 1) - ts
        above = above_blk + _xlane(c, ts) - _xlane(hr, ts)
        return bst, above

    def refine_level(bucket_fn, count, k, cur_d):
        zero_hist()
        nv = (count + (L - 1)) // L

        @plsc.parallel_loop(0, nv, step=1, unroll=2)
        def hst(i):
            sk = cand_v[pl.ds(i * L, L)]
            valid = (i * L + iota) < count
            plsc.addupdate_scatter(hist_v, [bucket_fn(sk)], ones16, mask=valid)

        bst, above = find_threshold(k)

        @plsc.parallel_loop(0, nv, step=1, unroll=2, carry=(cur_d - 1, zeros16 - 1))
        def flt(i, carry):
            cd, cc = carry
            sk = cand_v[pl.ds(i * L, L)]
            valid = (i * L + iota) < count
            b = bucket_fn(sk)
            gt = jnp.logical_and(valid, b > bst)
            eq = jnp.logical_and(valid, b == bst)
            pg = plsc.cumsum(ones16, mask=gt)
            plsc.store_scatter(def_v, [cd + pg], sk, mask=gt)
            pe = plsc.cumsum(ones16, mask=eq)
            plsc.store_scatter(cand_v, [cc + pe], sk, mask=eq)
            return (
                cd + plsc.all_reduce_population_count(gt),
                cc + plsc.all_reduce_population_count(eq),
            )
        cur_d, cur_c = flt
        return cur_d + 1, jnp.max(cur_c) + 1, bst, above

    def bucket_b1(sk):
        return lax.shift_right_arithmetic(sk, 24) + 128

    def bucket_b2(sk):
        return lax.shift_right_arithmetic(sk, 16) & jnp.int32(0xFF)

    def process_row(x_v, r, j):
        zero_hist()

        @plsc.parallel_loop(0, NVEC // SSTRIDE, step=1, unroll=4)
        def samp(i):
            x = x_v[pl.ds(i * (SSTRIDE * L), L)]
            sk = _keyize(lax.bitcast_convert_type(x, jnp.int32))
            plsc.addupdate_scatter(hist_v, [bucket_b1(sk)], ones16)
        bst_s, above_s = find_threshold(jnp.full((L,), SMIN, jnp.int32))
        zero_hist()

        @plsc.parallel_loop(0, NVEC // SSTRIDE, step=1, unroll=4)
        def samp2(i):
            x = x_v[pl.ds(i * (SSTRIDE * L), L)]
            sk = _keyize(lax.bitcast_convert_type(x, jnp.int32))
            m = bucket_b1(sk) == bst_s
            plsc.addupdate_scatter(hist_v, [bucket_b2(sk)], ones16, mask=m)
        bst2_s, _ = find_threshold(jnp.int32(SMIN) - above_s)
        tk = lax.shift_left(bst_s - 128, 24) | lax.shift_left(bst2_s, 16)
        t_low = lax.bitcast_convert_type(
            tk ^ (lax.shift_right_arithmetic(tk, 31) & _MASK31), jnp.float32)

        def compact_pass(t):
            @plsc.parallel_loop(0, NVEC, step=1, unroll=16, carry=zeros16 - 1)
            def pb(i, cc):
                x = x_v[pl.ds(i * L, L)]
                hot = x >= t
                p = plsc.cumsum(ones16, mask=hot)
                plsc.store_scatter(
                    cand_v, [cc + p],
                    lax.bitcast_convert_type(x, jnp.int32), mask=hot)
                return cc + plsc.all_reduce_population_count(hot)
            return jnp.max(pb) + 1

        def trip(s, carry):
            count, t = carry
            need = jnp.logical_or(s == 0, count < KTOP)
            count = lax.cond(need, lambda: compact_pass(t), lambda: count)
            return (count, jnp.full((L,), -jnp.inf, jnp.float32))
        count, _ = lax.fori_loop(0, 2, trip, (jnp.int32(0), t_low))

        @plsc.parallel_loop(0, (count + (L - 1)) // L, step=1, unroll=4)
        def kz(i):
            u = cand_v[pl.ds(i * L, L)]
            cand_v[pl.ds(i * L, L)] = _keyize(u)

        def lvl_body(lvl, carry):
            cur_d, cnt, k, tacc = carry
            shift = 24 - 8 * lvl
            bias = jnp.where(lvl == 0, jnp.int32(0x80), jnp.int32(0))

            def bucket_fn(sk):
                return (
                    lax.shift_right_arithmetic(sk, shift) & jnp.int32(0xFF)
                ) ^ bias
            cur_d, cnt2, bst, above = refine_level(bucket_fn, cnt, k, cur_d)
            tacc = tacc | lax.shift_left(bst ^ bias, shift)
            return (cur_d, cnt2, k - above, tacc)
        cur_d, _cn, k4, t_key = lax.fori_loop(
            0, 4, lvl_body,
            (zeros16, count, jnp.full((L,), KTOP, jnp.int32), zeros16))
        for t in range(4):
            m = (t * L + iota) < k4
            idx = cur_d + t * L + iota
            plsc.store_scatter(def_v, [idx], t_key, mask=m)

        d0 = def_v[pl.ds(0, L)]
        d1 = def_v[pl.ds(L, L)]
        d2 = def_v[pl.ds(2 * L, L)]
        d3 = def_v[pl.ds(3 * L, L)]
        e0, e1, e2, e3 = _sort64(d0, d1, d2, d3, iota)
        for t, e in enumerate((e3, e2, e1, e0)):
            w = lax.rev(e, (0,))
            u = w ^ (lax.shift_right_arithmetic(w, 31) & _MASK31)
            out_v[j, pl.ds(t * L, L)] = lax.bitcast_convert_type(
                u, jnp.float32)
        pltpu.async_copy(out_v.at[j], out_hbm.at[r], so)

    r0 = wid * RPW
    nhalf = RPW // 2
    pltpu.async_copy(x_hbm.at[r0], xa_v, sa)
    pltpu.async_copy(x_hbm.at[r0 + 1], xb_v, sb)

    def rows(jo, c):
        r = r0 + 2 * jo
        pltpu.make_async_copy(x_hbm.at[r], xa_v, sa).wait()
        process_row(xa_v, r, 2 * jo)

        @pl.when(jo < nhalf - 1)
        def _():
            pltpu.async_copy(x_hbm.at[r + 2], xa_v, sa)
        pltpu.make_async_copy(x_hbm.at[r + 1], xb_v, sb).wait()
        process_row(xb_v, r + 1, 2 * jo + 1)

        @pl.when(jo < nhalf - 1)
        def _():
            pltpu.async_copy(x_hbm.at[r + 3], xb_v, sb)
        return c
    lax.fori_loop(0, nhalf, rows, 0)
    for _ in range(RPW):
        pltpu.make_async_copy(out_v.at[0], out_hbm.at[r0], so).wait()


_topk_sc = functools.partial(
    pl.kernel,
    out_type=jax.ShapeDtypeStruct((ROWS, KTOP), jnp.float32),
    mesh=_MESH,
    compiler_params=pltpu.CompilerParams(needs_layout_passes=False),
    scratch_types=[
        pltpu.VMEM((COLS,), jnp.float32),
        pltpu.VMEM((COLS,), jnp.float32),
        pltpu.VMEM((COLS,), jnp.int32),
        pltpu.VMEM((NB,), jnp.int32),
        pltpu.VMEM((2 * KTOP,), jnp.int32),
        pltpu.VMEM((RPW, KTOP), jnp.float32),
        pltpu.SemaphoreType.DMA,
        pltpu.SemaphoreType.DMA,
        pltpu.SemaphoreType.DMA,
    ],
)(_body)


def kernel(input):
    return _topk_sc(input)
